# baseline (speedup 1.0000x reference)
import functools

import jax
import jax.numpy as jnp
from jax import lax
from jax.experimental import pallas as pl
from jax.experimental.pallas import tpu as pltpu
from jax.experimental.pallas import tpu_sc as plsc

N = 4096
S = 50
E = 64
H = 64
TOTAL = N * S

NC, NS = 2, 16
NW = NC * NS
ROWS_PER_W = TOTAL // NW
GROUP = 128
GROUPS = ROWS_PER_W // GROUP
GPC = 5
NCHUNK = GROUPS // GPC
CHUNK_ROWS = GPC * GROUP
VP = 125
EP = 128


def _make_gather():
  mesh = plsc.VectorSubcoreMesh(
      core_axis_name="c", subcore_axis_name="s",
      num_cores=NC, num_subcores=NS)

  @functools.partial(
      pl.kernel,
      mesh=mesh,
      out_type=jax.ShapeDtypeStruct((TOTAL, EP), jnp.float32),
      scratch_types=[
          pltpu.VMEM((GROUPS, GROUP), jnp.int32),
          pltpu.VMEM((GROUPS, GROUP), jnp.int32),
          pltpu.VMEM((CHUNK_ROWS, EP), jnp.float32),
          pltpu.SemaphoreType.DMA,
      ],
      compiler_params=pltpu.CompilerParams(use_tc_tiling_on_sc=True),
  )
  def gather(xflat_hbm, perm_hbm, table_hbm, out_hbm, perm_v, idx_v, rows_v,
             sem):
    wid = lax.axis_index("s") * NC + lax.axis_index("c")
    pltpu.sync_copy(perm_hbm.at[wid], perm_v)

    def idx_chunk(c, carry):
      for b in range(GPC):
        pltpu.make_async_copy(
            xflat_hbm.at[perm_v.at[c * GPC + b]],
            idx_v.at[c * GPC + b],
            sem).start()
      for b in range(GPC):
        pltpu.make_async_copy(
            xflat_hbm.at[perm_v.at[c * GPC + b]],
            idx_v.at[c * GPC + b],
            sem).wait()
      return carry

    lax.fori_loop(0, NCHUNK, idx_chunk, 0)

    def chunk_body(c, carry):
      for b in range(GPC):
        pltpu.make_async_copy(
            table_hbm.at[idx_v.at[c * GPC + b]],
            rows_v.at[pl.ds(b * GROUP, GROUP)],
            sem).start()
      for b in range(GPC):
        pltpu.make_async_copy(
            table_hbm.at[idx_v.at[c * GPC + b]],
            rows_v.at[pl.ds(b * GROUP, GROUP)],
            sem).wait()
      pltpu.sync_copy(
          rows_v,
          out_hbm.at[pl.ds(wid * ROWS_PER_W + c * CHUNK_ROWS, CHUNK_ROWS)])
      return carry

    lax.fori_loop(0, NCHUNK, chunk_body, 0)

  return gather


_gather_cache = []


def _get_gather():
  if not _gather_cache:
    _gather_cache.append(_make_gather())
  return _gather_cache[0]


def _lstm_body(x_ref, w_ref, b_ref, out_ref, h_s, c_s):
  t = pl.program_id(0)

  @pl.when(t == 0)
  def _():
    h_s[...] = jnp.zeros_like(h_s)
    c_s[...] = jnp.zeros_like(c_s)

  x = x_ref[0][:, :E]
  xh = jnp.concatenate([x, h_s[...]], axis=1)
  gates = jnp.dot(xh, w_ref[...],
                  preferred_element_type=jnp.float32) + b_ref[...]
  i = jax.nn.sigmoid(gates[:, 0 * H:1 * H])
  f = jax.nn.sigmoid(gates[:, 1 * H:2 * H])
  g = jnp.tanh(gates[:, 2 * H:3 * H])
  o = jax.nn.sigmoid(gates[:, 3 * H:4 * H])
  c = f * c_s[...] + i * g
  h = o * jnp.tanh(c)
  c_s[...] = c
  h_s[...] = h
  out_ref[0] = h


def _lstm(xs, w_cat, bias, interpret=False):
  return pl.pallas_call(
      _lstm_body,
      grid=(S,),
      in_specs=[
          pl.BlockSpec((1, N, EP), lambda t: (t, 0, 0)),
          pl.BlockSpec((2 * H, 4 * H), lambda t: (0, 0)),
          pl.BlockSpec((1, 4 * H), lambda t: (0, 0)),
      ],
      out_specs=pl.BlockSpec((1, N, H), lambda t: (t, 0, 0)),
      out_shape=jax.ShapeDtypeStruct((S, N, H), jnp.float32),
      scratch_shapes=[
          pltpu.VMEM((N, H), jnp.float32),
          pltpu.VMEM((N, H), jnp.float32),
      ],
      compiler_params=pltpu.CompilerParams(
          vmem_limit_bytes=100 * 1024 * 1024),
      interpret=interpret,
  )(xs, w_cat, bias)


def kernel(X, emb_table, W_ih, W_hh, b_ih, b_hh):
  w_cat = jnp.concatenate([W_ih.T, W_hh.T], axis=0)
  bias = (b_ih + b_hh)[None, :]
  r = jnp.arange(TOTAL, dtype=jnp.int32)
  perm = ((r & (N - 1)) * S + (r >> 12)).reshape(NW, GROUPS, GROUP)
  xflat = X.reshape(TOTAL)
  tblp = jnp.pad(emb_table, ((0, 4), (0, EP - E)))
  wordemb = _get_gather()(xflat, perm, tblp)
  xs = wordemb.reshape(S, N, EP)
  out_tm = _lstm(xs, w_cat, bias)
  h = out_tm[S - 1][None]
  out = jnp.transpose(out_tm, (1, 0, 2))
  return (h, out)


# speedup vs baseline: 1.0834x; 1.0834x over previous
import functools

import jax
import jax.numpy as jnp
from jax import lax
from jax.experimental import pallas as pl
from jax.experimental.pallas import tpu as pltpu
from jax.experimental.pallas import tpu_sc as plsc

N = 4096
S = 50
E = 64
H = 64
TOTAL = N * S

NC, NS = 2, 16
NW = NC * NS
ROWS_PER_W = TOTAL // NW
GROUP = 128
GROUPS = ROWS_PER_W // GROUP
GPC = 5
NCHUNK = GROUPS // GPC
CHUNK_ROWS = GPC * GROUP
VP = 125
EP = 128


def _make_gather():
  mesh = plsc.VectorSubcoreMesh(
      core_axis_name="c", subcore_axis_name="s",
      num_cores=NC, num_subcores=NS)

  @functools.partial(
      pl.kernel,
      mesh=mesh,
      out_type=jax.ShapeDtypeStruct((TOTAL, EP), jnp.float32),
      scratch_types=[
          pltpu.VMEM((GROUPS, GROUP), jnp.int32),
          pltpu.VMEM((GROUPS, GROUP), jnp.int32),
          pltpu.VMEM((CHUNK_ROWS, EP), jnp.float32),
          pltpu.SemaphoreType.DMA,
      ],
      compiler_params=pltpu.CompilerParams(use_tc_tiling_on_sc=True),
  )
  def gather(xflat_hbm, perm_hbm, table_hbm, out_hbm, perm_v, idx_v, rows_v,
             sem):
    wid = lax.axis_index("s") * NC + lax.axis_index("c")
    pltpu.sync_copy(perm_hbm.at[wid], perm_v)

    def idx_chunk(c, carry):
      for b in range(GPC):
        pltpu.make_async_copy(
            xflat_hbm.at[perm_v.at[c * GPC + b]],
            idx_v.at[c * GPC + b],
            sem).start()
      for b in range(GPC):
        pltpu.make_async_copy(
            xflat_hbm.at[perm_v.at[c * GPC + b]],
            idx_v.at[c * GPC + b],
            sem).wait()
      return carry

    lax.fori_loop(0, NCHUNK, idx_chunk, 0)

    def chunk_body(c, carry):
      for b in range(GPC):
        pltpu.make_async_copy(
            table_hbm.at[idx_v.at[c * GPC + b]],
            rows_v.at[pl.ds(b * GROUP, GROUP)],
            sem).start()
      for b in range(GPC):
        pltpu.make_async_copy(
            table_hbm.at[idx_v.at[c * GPC + b]],
            rows_v.at[pl.ds(b * GROUP, GROUP)],
            sem).wait()
      pltpu.sync_copy(
          rows_v,
          out_hbm.at[pl.ds(wid * ROWS_PER_W + c * CHUNK_ROWS, CHUNK_ROWS)])
      return carry

    lax.fori_loop(0, NCHUNK, chunk_body, 0)

  return gather


_gather_cache = []


def _get_gather():
  if not _gather_cache:
    _gather_cache.append(_make_gather())
  return _gather_cache[0]


def _lstm_body(x_ref, w_ref, b_ref, out_ref, h_s, c_s):
  t = pl.program_id(0)

  @pl.when(t == 0)
  def _():
    h_s[...] = jnp.zeros_like(h_s)
    c_s[...] = jnp.zeros_like(c_s)

  x = x_ref[0][:, :E]
  xh = jnp.concatenate([x, h_s[...]], axis=1)
  gates = jnp.dot(xh, w_ref[...],
                  preferred_element_type=jnp.float32) + b_ref[...]
  i = jax.nn.sigmoid(gates[:, 0 * H:1 * H])
  f = jax.nn.sigmoid(gates[:, 1 * H:2 * H])
  g = jnp.tanh(gates[:, 2 * H:3 * H])
  o = jax.nn.sigmoid(gates[:, 3 * H:4 * H])
  c = f * c_s[...] + i * g
  h = o * jnp.tanh(c)
  c_s[...] = c
  h_s[...] = h
  out_ref[0] = h


def _lstm(xs, w_cat, bias, interpret=False):
  return pl.pallas_call(
      _lstm_body,
      grid=(S,),
      in_specs=[
          pl.BlockSpec((1, N, EP), lambda t: (t, 0, 0)),
          pl.BlockSpec((2 * H, 4 * H), lambda t: (0, 0)),
          pl.BlockSpec((1, 4 * H), lambda t: (0, 0)),
      ],
      out_specs=pl.BlockSpec((1, N, H), lambda t: (t, 0, 0)),
      out_shape=jax.ShapeDtypeStruct((S, N, H), jnp.float32),
      scratch_shapes=[
          pltpu.VMEM((N, H), jnp.float32),
          pltpu.VMEM((N, H), jnp.float32),
      ],
      compiler_params=pltpu.CompilerParams(
          vmem_limit_bytes=100 * 1024 * 1024),
      interpret=interpret,
  )(xs, w_cat, bias)


VROWS = 1000004
VPADR = VROWS + 4
TBC = 2048
TGRID = (VROWS + TBC - 1) // TBC


def _tpad_body(x_ref, o_ref):
  y = x_ref[...].T
  o_ref[:, :E] = y
  o_ref[:, E:] = jnp.zeros_like(y)


def _transpose_pad(tt):
  return pl.pallas_call(
      _tpad_body,
      grid=(TGRID,),
      in_specs=[pl.BlockSpec((E, TBC), lambda c: (0, c))],
      out_specs=pl.BlockSpec((TBC, EP), lambda c: (c, 0)),
      out_shape=jax.ShapeDtypeStruct((VPADR, EP), jnp.float32),
  )(tt)


def kernel(X, emb_table, W_ih, W_hh, b_ih, b_hh):
  w_cat = jnp.concatenate([W_ih.T, W_hh.T], axis=0)
  bias = (b_ih + b_hh)[None, :]
  r = jnp.arange(TOTAL, dtype=jnp.int32)
  perm = ((r & (N - 1)) * S + (r >> 12)).reshape(NW, GROUPS, GROUP)
  xflat = X.reshape(TOTAL)
  tblp = _transpose_pad(emb_table.T)
  wordemb = _get_gather()(xflat, perm, tblp)
  xs = wordemb.reshape(S, N, EP)
  out_tm = _lstm(xs, w_cat, bias)
  h = out_tm[S - 1][None]
  out = jnp.transpose(out_tm, (1, 0, 2))
  return (h, out)


# speedup vs baseline: 1.1005x; 1.0158x over previous
import functools

import jax
import jax.numpy as jnp
from jax import lax
from jax.experimental import pallas as pl
from jax.experimental.pallas import tpu as pltpu
from jax.experimental.pallas import tpu_sc as plsc

N = 4096
S = 50
E = 64
H = 64
TOTAL = N * S

NC, NS = 2, 16
NW = NC * NS
ROWS_PER_W = TOTAL // NW
GROUP = 128
GROUPS = ROWS_PER_W // GROUP
GPC = 5
NCHUNK = GROUPS // GPC
CHUNK_ROWS = GPC * GROUP
VP = 125
EP = 128


def _make_gather():
  mesh = plsc.VectorSubcoreMesh(
      core_axis_name="c", subcore_axis_name="s",
      num_cores=NC, num_subcores=NS)

  @functools.partial(
      pl.kernel,
      mesh=mesh,
      out_type=jax.ShapeDtypeStruct((TOTAL, EP), jnp.float32),
      scratch_types=[
          pltpu.VMEM((GROUPS, GROUP), jnp.int32),
          pltpu.VMEM((GROUPS, GROUP), jnp.int32),
          pltpu.VMEM((CHUNK_ROWS, EP), jnp.float32),
          pltpu.SemaphoreType.DMA,
      ],
      compiler_params=pltpu.CompilerParams(use_tc_tiling_on_sc=True),
  )
  def gather(xflat_hbm, perm_hbm, table_hbm, out_hbm, perm_v, idx_v, rows_v,
             sem):
    wid = lax.axis_index("s") * NC + lax.axis_index("c")
    pltpu.sync_copy(perm_hbm.at[wid], perm_v)

    def idx_chunk(c, carry):
      for b in range(GPC):
        pltpu.make_async_copy(
            xflat_hbm.at[perm_v.at[c * GPC + b]],
            idx_v.at[c * GPC + b],
            sem).start()
      for b in range(GPC):
        pltpu.make_async_copy(
            xflat_hbm.at[perm_v.at[c * GPC + b]],
            idx_v.at[c * GPC + b],
            sem).wait()
      return carry

    lax.fori_loop(0, NCHUNK, idx_chunk, 0)

    def chunk_body(c, carry):
      for b in range(GPC):
        pltpu.make_async_copy(
            table_hbm.at[idx_v.at[c * GPC + b]],
            rows_v.at[pl.ds(b * GROUP, GROUP)],
            sem).start()
      for b in range(GPC):
        pltpu.make_async_copy(
            table_hbm.at[idx_v.at[c * GPC + b]],
            rows_v.at[pl.ds(b * GROUP, GROUP)],
            sem).wait()
      pltpu.sync_copy(
          rows_v,
          out_hbm.at[pl.ds(wid * ROWS_PER_W + c * CHUNK_ROWS, CHUNK_ROWS)])
      return carry

    lax.fori_loop(0, NCHUNK, chunk_body, 0)

  return gather


_gather_cache = []


def _get_gather():
  if not _gather_cache:
    _gather_cache.append(_make_gather())
  return _gather_cache[0]


def _lstm_body(x_ref, w_ref, b_ref, out_ref, h_s, c_s):
  t = pl.program_id(0)

  @pl.when(t == 0)
  def _():
    h_s[...] = jnp.zeros_like(h_s)
    c_s[...] = jnp.zeros_like(c_s)

  x = x_ref[0][:, :E]
  xh = jnp.concatenate([x, h_s[...]], axis=1)
  gates = jnp.dot(xh, w_ref[...],
                  preferred_element_type=jnp.float32) + b_ref[...]
  i = jax.nn.sigmoid(gates[:, 0 * H:1 * H])
  f = jax.nn.sigmoid(gates[:, 1 * H:2 * H])
  g = jnp.tanh(gates[:, 2 * H:3 * H])
  o = jax.nn.sigmoid(gates[:, 3 * H:4 * H])
  c = f * c_s[...] + i * g
  h = o * jnp.tanh(c)
  c_s[...] = c
  h_s[...] = h
  out_ref[0] = h.T


def _lstm(xs, w_cat, bias, interpret=False):
  return pl.pallas_call(
      _lstm_body,
      grid=(S,),
      in_specs=[
          pl.BlockSpec((1, N, EP), lambda t: (t, 0, 0)),
          pl.BlockSpec((2 * H, 4 * H), lambda t: (0, 0)),
          pl.BlockSpec((1, 4 * H), lambda t: (0, 0)),
      ],
      out_specs=pl.BlockSpec((1, H, N), lambda t: (t, 0, 0)),
      out_shape=jax.ShapeDtypeStruct((S, H, N), jnp.float32),
      scratch_shapes=[
          pltpu.VMEM((N, H), jnp.float32),
          pltpu.VMEM((N, H), jnp.float32),
      ],
      compiler_params=pltpu.CompilerParams(
          vmem_limit_bytes=100 * 1024 * 1024),
      interpret=interpret,
  )(xs, w_cat, bias)


VROWS = 1000004
VPADR = VROWS + 4
TBC = 2048
TGRID = (VROWS + TBC - 1) // TBC


def _tpad_body(x_ref, o_ref):
  y = x_ref[...].T
  o_ref[:, :E] = y
  o_ref[:, E:] = jnp.zeros_like(y)


def _transpose_pad(tt):
  return pl.pallas_call(
      _tpad_body,
      grid=(TGRID,),
      in_specs=[pl.BlockSpec((E, TBC), lambda c: (0, c))],
      out_specs=pl.BlockSpec((TBC, EP), lambda c: (c, 0)),
      out_shape=jax.ShapeDtypeStruct((VPADR, EP), jnp.float32),
  )(tt)


def kernel(X, emb_table, W_ih, W_hh, b_ih, b_hh):
  w_cat = jnp.concatenate([W_ih.T, W_hh.T], axis=0)
  bias = (b_ih + b_hh)[None, :]
  r = jnp.arange(TOTAL, dtype=jnp.int32)
  perm = ((r & (N - 1)) * S + (r >> 12)).reshape(NW, GROUPS, GROUP)
  xflat = X.reshape(TOTAL)
  tblp = _transpose_pad(emb_table.T)
  wordemb = _get_gather()(xflat, perm, tblp)
  xs = wordemb.reshape(S, N, EP)
  out_thn = _lstm(xs, w_cat, bias)
  h = out_thn[S - 1].T[None]
  out = jnp.transpose(out_thn, (2, 0, 1))
  return (h, out)


# speedup vs baseline: 1.4938x; 1.3574x over previous
import functools

import jax
import jax.numpy as jnp
from jax import lax
from jax.experimental import pallas as pl
from jax.experimental.pallas import tpu as pltpu
from jax.experimental.pallas import tpu_sc as plsc

N = 4096
S = 50
E = 64
H = 64
TOTAL = N * S

NC, NS = 2, 16
NW = NC * NS
ROWS_PER_W = TOTAL // NW
GROUP = 128
GROUPS = ROWS_PER_W // GROUP
GPC = 5
NCHUNK = GROUPS // GPC
CHUNK_ROWS = GPC * GROUP
VP = 125
EP = 128


def _make_gather():
  mesh = plsc.VectorSubcoreMesh(
      core_axis_name="c", subcore_axis_name="s",
      num_cores=NC, num_subcores=NS)

  @functools.partial(
      pl.kernel,
      mesh=mesh,
      out_type=jax.ShapeDtypeStruct((TOTAL, EP), jnp.float32),
      scratch_types=[
          pltpu.VMEM((GROUPS, GROUP), jnp.int32),
          pltpu.VMEM((GROUPS, GROUP), jnp.int32),
          pltpu.VMEM((CHUNK_ROWS, EP), jnp.float32),
          pltpu.SemaphoreType.DMA,
      ],
      compiler_params=pltpu.CompilerParams(use_tc_tiling_on_sc=True),
  )
  def gather(xflat_hbm, perm_hbm, table_hbm, out_hbm, perm_v, idx_v, rows_v,
             sem):
    wid = lax.axis_index("s") * NC + lax.axis_index("c")
    pltpu.sync_copy(perm_hbm.at[wid], perm_v)

    def idx_chunk(c, carry):
      for b in range(GPC):
        pltpu.make_async_copy(
            xflat_hbm.at[perm_v.at[c * GPC + b]],
            idx_v.at[c * GPC + b],
            sem).start()
      for b in range(GPC):
        pltpu.make_async_copy(
            xflat_hbm.at[perm_v.at[c * GPC + b]],
            idx_v.at[c * GPC + b],
            sem).wait()
      return carry

    lax.fori_loop(0, NCHUNK, idx_chunk, 0)

    def chunk_body(c, carry):
      for b in range(GPC):
        pltpu.make_async_copy(
            table_hbm.at[idx_v.at[c * GPC + b]],
            rows_v.at[pl.ds(b * GROUP, GROUP)],
            sem).start()
      for b in range(GPC):
        pltpu.make_async_copy(
            table_hbm.at[idx_v.at[c * GPC + b]],
            rows_v.at[pl.ds(b * GROUP, GROUP)],
            sem).wait()
      pltpu.sync_copy(
          rows_v,
          out_hbm.at[pl.ds(wid * ROWS_PER_W + c * CHUNK_ROWS, CHUNK_ROWS)])
      return carry

    lax.fori_loop(0, NCHUNK, chunk_body, 0)

  return gather


_gather_cache = []


def _get_gather():
  if not _gather_cache:
    _gather_cache.append(_make_gather())
  return _gather_cache[0]


def _lstm_body(x_ref, w_ref, b_ref, out_ref, h_s, c_s):
  t = pl.program_id(0)

  @pl.when(t == 0)
  def _():
    h_s[...] = jnp.zeros_like(h_s)
    c_s[...] = jnp.zeros_like(c_s)

  x = x_ref[0][:, :E]
  xh = jnp.concatenate([x, h_s[...]], axis=1)
  gates = jnp.dot(xh, w_ref[...],
                  preferred_element_type=jnp.float32) + b_ref[...]
  i = jax.nn.sigmoid(gates[:, 0 * H:1 * H])
  f = jax.nn.sigmoid(gates[:, 1 * H:2 * H])
  g = jnp.tanh(gates[:, 2 * H:3 * H])
  o = jax.nn.sigmoid(gates[:, 3 * H:4 * H])
  c = f * c_s[...] + i * g
  h = o * jnp.tanh(c)
  c_s[...] = c
  h_s[...] = h
  out_ref[0] = h.T


def _lstm(xs, w_cat, bias, interpret=False):
  return pl.pallas_call(
      _lstm_body,
      grid=(S,),
      in_specs=[
          pl.BlockSpec((1, N, EP), lambda t: (t, 0, 0)),
          pl.BlockSpec((2 * H, 4 * H), lambda t: (0, 0)),
          pl.BlockSpec((1, 4 * H), lambda t: (0, 0)),
      ],
      out_specs=pl.BlockSpec((1, H, N), lambda t: (t, 0, 0)),
      out_shape=jax.ShapeDtypeStruct((S, H, N), jnp.float32),
      scratch_shapes=[
          pltpu.VMEM((N, H), jnp.float32),
          pltpu.VMEM((N, H), jnp.float32),
      ],
      compiler_params=pltpu.CompilerParams(
          vmem_limit_bytes=100 * 1024 * 1024),
      interpret=interpret,
  )(xs, w_cat, bias)


VROWS = 1000004
VPADR = VROWS + 4
TBC = 8192
TGRID = (VROWS + TBC - 1) // TBC


def _tpad_body(x_ref, o_ref):
  y = x_ref[...].T
  o_ref[:, :E] = y
  o_ref[:, E:] = jnp.zeros_like(y)


def _transpose_pad(tt):
  return pl.pallas_call(
      _tpad_body,
      grid=(TGRID,),
      in_specs=[pl.BlockSpec((E, TBC), lambda c: (0, c))],
      out_specs=pl.BlockSpec((TBC, EP), lambda c: (c, 0)),
      out_shape=jax.ShapeDtypeStruct((VPADR, EP), jnp.float32),
  )(tt)


def kernel(X, emb_table, W_ih, W_hh, b_ih, b_hh):
  w_cat = jnp.concatenate([W_ih.T, W_hh.T], axis=0)
  bias = (b_ih + b_hh)[None, :]
  r = jnp.arange(TOTAL, dtype=jnp.int32)
  perm = ((r & (N - 1)) * S + (r >> 12)).reshape(NW, GROUPS, GROUP)
  xflat = X.reshape(TOTAL)
  tblp = _transpose_pad(emb_table.T)
  wordemb = _get_gather()(xflat, perm, tblp)
  xs = wordemb.reshape(S, N, EP)
  out_thn = _lstm(xs, w_cat, bias)
  h = out_thn[S - 1].T[None]
  out = jnp.transpose(out_thn, (2, 0, 1))
  return (h, out)


# speedup vs baseline: 1.5451x; 1.0343x over previous
import functools

import jax
import jax.numpy as jnp
from jax import lax
from jax.experimental import pallas as pl
from jax.experimental.pallas import tpu as pltpu
from jax.experimental.pallas import tpu_sc as plsc

N = 4096
S = 50
E = 64
H = 64
TOTAL = N * S

NC, NS = 2, 16
NW = NC * NS
ROWS_PER_W = TOTAL // NW
GROUP = 128
GROUPS = ROWS_PER_W // GROUP
GPC = 5
NCHUNK = GROUPS // GPC
CHUNK_ROWS = GPC * GROUP
VP = 125
EP = 128


def _make_gather():
  mesh = plsc.VectorSubcoreMesh(
      core_axis_name="c", subcore_axis_name="s",
      num_cores=NC, num_subcores=NS)

  @functools.partial(
      pl.kernel,
      mesh=mesh,
      out_type=jax.ShapeDtypeStruct((TOTAL, EP), jnp.float32),
      scratch_types=[
          pltpu.VMEM((GROUPS, GROUP), jnp.int32),
          pltpu.VMEM((GROUPS, GROUP), jnp.int32),
          pltpu.VMEM((CHUNK_ROWS, EP), jnp.float32),
          pltpu.SemaphoreType.DMA,
      ],
      compiler_params=pltpu.CompilerParams(use_tc_tiling_on_sc=True),
  )
  def gather(xflat_hbm, perm_hbm, table_hbm, out_hbm, perm_v, idx_v, rows_v,
             sem):
    wid = lax.axis_index("s") * NC + lax.axis_index("c")
    pltpu.sync_copy(perm_hbm.at[wid], perm_v)

    def idx_chunk(c, carry):
      for b in range(GPC):
        pltpu.make_async_copy(
            xflat_hbm.at[perm_v.at[c * GPC + b]],
            idx_v.at[c * GPC + b],
            sem).start()
      for b in range(GPC):
        pltpu.make_async_copy(
            xflat_hbm.at[perm_v.at[c * GPC + b]],
            idx_v.at[c * GPC + b],
            sem).wait()
      return carry

    lax.fori_loop(0, NCHUNK, idx_chunk, 0)

    def chunk_body(c, carry):
      for b in range(GPC):
        pltpu.make_async_copy(
            table_hbm.at[idx_v.at[c * GPC + b]],
            rows_v.at[pl.ds(b * GROUP, GROUP)],
            sem).start()
      for b in range(GPC):
        pltpu.make_async_copy(
            table_hbm.at[idx_v.at[c * GPC + b]],
            rows_v.at[pl.ds(b * GROUP, GROUP)],
            sem).wait()
      pltpu.sync_copy(
          rows_v,
          out_hbm.at[pl.ds(wid * ROWS_PER_W + c * CHUNK_ROWS, CHUNK_ROWS)])
      return carry

    lax.fori_loop(0, NCHUNK, chunk_body, 0)

  return gather


_gather_cache = []


def _get_gather():
  if not _gather_cache:
    _gather_cache.append(_make_gather())
  return _gather_cache[0]


def _lstm_body(x_ref, w_ref, b_ref, out_ref, h_s, c_s):
  t = pl.program_id(0)

  @pl.when(t == 0)
  def _():
    h_s[...] = jnp.zeros_like(h_s)
    c_s[...] = jnp.zeros_like(c_s)

  x = x_ref[0][:, :E]
  xh = jnp.concatenate([x, h_s[...]], axis=1)
  gates = jnp.dot(xh, w_ref[...],
                  preferred_element_type=jnp.float32) + b_ref[...]
  i = jax.nn.sigmoid(gates[:, 0 * H:1 * H])
  f = jax.nn.sigmoid(gates[:, 1 * H:2 * H])
  g = jnp.tanh(gates[:, 2 * H:3 * H])
  o = jax.nn.sigmoid(gates[:, 3 * H:4 * H])
  c = f * c_s[...] + i * g
  h = o * jnp.tanh(c)
  c_s[...] = c
  h_s[...] = h
  out_ref[0] = h.T


def _lstm(xs, w_cat, bias, interpret=False):
  return pl.pallas_call(
      _lstm_body,
      grid=(S,),
      in_specs=[
          pl.BlockSpec((1, N, EP), lambda t: (t, 0, 0)),
          pl.BlockSpec((2 * H, 4 * H), lambda t: (0, 0)),
          pl.BlockSpec((1, 4 * H), lambda t: (0, 0)),
      ],
      out_specs=pl.BlockSpec((1, H, N), lambda t: (t, 0, 0)),
      out_shape=jax.ShapeDtypeStruct((S, H, N), jnp.float32),
      scratch_shapes=[
          pltpu.VMEM((N, H), jnp.float32),
          pltpu.VMEM((N, H), jnp.float32),
      ],
      compiler_params=pltpu.CompilerParams(
          vmem_limit_bytes=100 * 1024 * 1024),
      interpret=interpret,
  )(xs, w_cat, bias)


VROWS = 1000004
VPADR = VROWS + 4
TBC = 16384
TGRID = (VROWS + TBC - 1) // TBC


def _tpad_body(x_ref, o_ref):
  y = x_ref[...].T
  o_ref[:, :E] = y
  o_ref[:, E:] = jnp.zeros_like(y)


def _transpose_pad(tt):
  return pl.pallas_call(
      _tpad_body,
      grid=(TGRID,),
      in_specs=[pl.BlockSpec((E, TBC), lambda c: (0, c))],
      out_specs=pl.BlockSpec((TBC, EP), lambda c: (c, 0)),
      out_shape=jax.ShapeDtypeStruct((VPADR, EP), jnp.float32),
  )(tt)


def kernel(X, emb_table, W_ih, W_hh, b_ih, b_hh):
  w_cat = jnp.concatenate([W_ih.T, W_hh.T], axis=0)
  bias = (b_ih + b_hh)[None, :]
  r = jnp.arange(TOTAL, dtype=jnp.int32)
  perm = ((r & (N - 1)) * S + (r >> 12)).reshape(NW, GROUPS, GROUP)
  xflat = X.reshape(TOTAL)
  tblp = _transpose_pad(emb_table.T)
  wordemb = _get_gather()(xflat, perm, tblp)
  xs = wordemb.reshape(S, N, EP)
  out_thn = _lstm(xs, w_cat, bias)
  h = out_thn[S - 1].T[None]
  out = jnp.transpose(out_thn, (2, 0, 1))
  return (h, out)


# speedup vs baseline: 1.5627x; 1.0114x over previous
import functools

import jax
import jax.numpy as jnp
from jax import lax
from jax.experimental import pallas as pl
from jax.experimental.pallas import tpu as pltpu
from jax.experimental.pallas import tpu_sc as plsc

N = 4096
S = 50
E = 64
H = 64
TOTAL = N * S

NC, NS = 2, 16
NW = NC * NS
ROWS_PER_W = TOTAL // NW
GROUP = 128
GROUPS = ROWS_PER_W // GROUP
GPC = 5
NCHUNK = GROUPS // GPC
CHUNK_ROWS = GPC * GROUP
VP = 125
EP = 128


def _make_gather():
  mesh = plsc.VectorSubcoreMesh(
      core_axis_name="c", subcore_axis_name="s",
      num_cores=NC, num_subcores=NS)

  @functools.partial(
      pl.kernel,
      mesh=mesh,
      out_type=jax.ShapeDtypeStruct((TOTAL, EP), jnp.float32),
      scratch_types=[
          pltpu.VMEM((GROUPS, GROUP), jnp.int32),
          pltpu.VMEM((GROUPS, GROUP), jnp.int32),
          pltpu.VMEM((CHUNK_ROWS, EP), jnp.float32),
          pltpu.SemaphoreType.DMA,
      ],
      compiler_params=pltpu.CompilerParams(use_tc_tiling_on_sc=True),
  )
  def gather(xflat_hbm, perm_hbm, table_hbm, out_hbm, perm_v, idx_v, rows_v,
             sem):
    wid = lax.axis_index("s") * NC + lax.axis_index("c")
    pltpu.sync_copy(perm_hbm.at[wid], perm_v)

    def idx_chunk(c, carry):
      for b in range(GPC):
        pltpu.make_async_copy(
            xflat_hbm.at[perm_v.at[c * GPC + b]],
            idx_v.at[c * GPC + b],
            sem).start()
      for b in range(GPC):
        pltpu.make_async_copy(
            xflat_hbm.at[perm_v.at[c * GPC + b]],
            idx_v.at[c * GPC + b],
            sem).wait()
      return carry

    lax.fori_loop(0, NCHUNK, idx_chunk, 0)

    def chunk_body(c, carry):
      for b in range(GPC):
        pltpu.make_async_copy(
            table_hbm.at[idx_v.at[c * GPC + b]],
            rows_v.at[pl.ds(b * GROUP, GROUP)],
            sem).start()
      for b in range(GPC):
        pltpu.make_async_copy(
            table_hbm.at[idx_v.at[c * GPC + b]],
            rows_v.at[pl.ds(b * GROUP, GROUP)],
            sem).wait()
      pltpu.sync_copy(
          rows_v,
          out_hbm.at[pl.ds(wid * ROWS_PER_W + c * CHUNK_ROWS, CHUNK_ROWS)])
      return carry

    lax.fori_loop(0, NCHUNK, chunk_body, 0)

  return gather


_gather_cache = []


def _get_gather():
  if not _gather_cache:
    _gather_cache.append(_make_gather())
  return _gather_cache[0]


def _lstm_body(x_ref, w_ref, b_ref, out_ref, h_s, c_s):
  t = pl.program_id(0)

  @pl.when(t == 0)
  def _():
    h_s[...] = jnp.zeros_like(h_s)
    c_s[...] = jnp.zeros_like(c_s)

  x = x_ref[0][:, :E]
  xh = jnp.concatenate([x, h_s[...]], axis=1)
  gates = jnp.dot(xh, w_ref[...],
                  preferred_element_type=jnp.float32) + b_ref[...]
  i = jax.nn.sigmoid(gates[:, 0 * H:1 * H])
  f = jax.nn.sigmoid(gates[:, 1 * H:2 * H])
  g = jnp.tanh(gates[:, 2 * H:3 * H])
  o = jax.nn.sigmoid(gates[:, 3 * H:4 * H])
  c = f * c_s[...] + i * g
  h = o * jnp.tanh(c)
  c_s[...] = c
  h_s[...] = h
  out_ref[0] = h.T


def _lstm(xs, w_cat, bias, interpret=False):
  return pl.pallas_call(
      _lstm_body,
      grid=(S,),
      in_specs=[
          pl.BlockSpec((1, N, EP), lambda t: (t, 0, 0)),
          pl.BlockSpec((2 * H, 4 * H), lambda t: (0, 0)),
          pl.BlockSpec((1, 4 * H), lambda t: (0, 0)),
      ],
      out_specs=pl.BlockSpec((1, H, N), lambda t: (t, 0, 0)),
      out_shape=jax.ShapeDtypeStruct((S, H, N), jnp.float32),
      scratch_shapes=[
          pltpu.VMEM((N, H), jnp.float32),
          pltpu.VMEM((N, H), jnp.float32),
      ],
      compiler_params=pltpu.CompilerParams(
          vmem_limit_bytes=100 * 1024 * 1024),
      interpret=interpret,
  )(xs, w_cat, bias)


VROWS = 1000004
VPADR = VROWS + 4
TBC = 32768
TGRID = (VROWS + TBC - 1) // TBC


def _tpad_body(x_ref, o_ref):
  y = x_ref[...].T
  o_ref[:, :E] = y
  o_ref[:, E:] = jnp.zeros_like(y)


def _transpose_pad(tt):
  return pl.pallas_call(
      _tpad_body,
      grid=(TGRID,),
      in_specs=[pl.BlockSpec((E, TBC), lambda c: (0, c))],
      out_specs=pl.BlockSpec((TBC, EP), lambda c: (c, 0)),
      out_shape=jax.ShapeDtypeStruct((VPADR, EP), jnp.float32),
      compiler_params=pltpu.CompilerParams(
          vmem_limit_bytes=100 * 1024 * 1024),
  )(tt)


def kernel(X, emb_table, W_ih, W_hh, b_ih, b_hh):
  w_cat = jnp.concatenate([W_ih.T, W_hh.T], axis=0)
  bias = (b_ih + b_hh)[None, :]
  r = jnp.arange(TOTAL, dtype=jnp.int32)
  perm = ((r & (N - 1)) * S + (r >> 12)).reshape(NW, GROUPS, GROUP)
  xflat = X.reshape(TOTAL)
  tblp = _transpose_pad(emb_table.T)
  wordemb = _get_gather()(xflat, perm, tblp)
  xs = wordemb.reshape(S, N, EP)
  out_thn = _lstm(xs, w_cat, bias)
  h = out_thn[S - 1].T[None]
  out = jnp.transpose(out_thn, (2, 0, 1))
  return (h, out)
